# baseline (speedup 1.0000x reference)
import functools
import math

import jax
import jax.numpy as jnp
from jax import lax
from jax.experimental import pallas as pl
from jax.experimental.pallas import tpu as pltpu
from jax.experimental.pallas import tpu_sc as plsc

_B, _S, _DIM, _H, _HD = 1, 2048, 1024, 16, 64
_E, _TOPK, _HID, _SHID = 8, 2, 1024, 1024
_EPS = 1e-6

_SBLK = 256
_ABLK = 512
_FBLK = 128
_NA = _TOPK * _S
_NPAD = _NA + _E * _FBLK
_NFB = _NPAD // _FBLK


def _rms(x, w):
    return x * lax.rsqrt(jnp.mean(x * x, axis=-1, keepdims=True) + _EPS) * w


def _dot_t(a, b):
    return lax.dot_general(a, b, (((1,), (1,)), ((), ())),
                           preferred_element_type=jnp.float32)


def _rope(y, cc, d1, d2):
    left = jnp.concatenate([y[:, 1:], y[:, :1]], axis=1)
    right = jnp.concatenate([y[:, -1:], y[:, :-1]], axis=1)
    return y * cc + left * d1 + right * d2


def _qkv_body(x_ref, c_ref, d1_ref, d2_ref, anw_ref, wq_ref, wk_ref, wv_ref,
              q_ref, k_ref, v_ref):
    xn = _rms(x_ref[...], anw_ref[...]).astype(jnp.bfloat16)
    cc = jnp.concatenate([c_ref[...]] * (_DIM // 128), axis=1)
    d1 = jnp.concatenate([d1_ref[...]] * (_DIM // 128), axis=1)
    d2 = jnp.concatenate([d2_ref[...]] * (_DIM // 128), axis=1)
    q = _dot_t(xn, wq_ref[...])
    k = _dot_t(xn, wk_ref[...])
    q_ref[...] = _rope(q, cc, d1, d2).astype(jnp.bfloat16)
    k_ref[...] = _rope(k, cc, d1, d2).astype(jnp.bfloat16)
    v_ref[...] = _dot_t(xn, wv_ref[...]).astype(jnp.bfloat16)


def _qkv_call(x2, c_tab, d1_tab, d2_tab, anw, wqb, wkb, wvb):
    n = _S // _SBLK
    blk_s = pl.BlockSpec((_SBLK, _DIM), lambda i: (i, 0))
    blk_t = pl.BlockSpec((_SBLK, 128), lambda i: (i, 0))
    w_full = pl.BlockSpec((_DIM, _DIM), lambda i: (0, 0))
    return pl.pallas_call(
        _qkv_body,
        grid=(n,),
        in_specs=[blk_s, blk_t, blk_t, blk_t,
                  pl.BlockSpec((1, _DIM), lambda i: (0, 0)),
                  w_full, w_full, w_full],
        out_specs=[blk_s, blk_s, blk_s],
        out_shape=[jax.ShapeDtypeStruct((_S, _DIM), jnp.bfloat16)] * 3,
    )(x2, c_tab, d1_tab, d2_tab, anw, wqb, wkb, wvb)


def _att_body(q_ref, k_ref, v_ref, o_ref):
    ones = jnp.ones((_S, 1), jnp.bfloat16)
    outs = []
    for p in range(2):
        q = q_ref[:, p * _HD:(p + 1) * _HD]
        k = k_ref[:, p * _HD:(p + 1) * _HD]
        v = v_ref[:, p * _HD:(p + 1) * _HD]
        s = _dot_t(q, k) * (1.0 / math.sqrt(_HD))
        eb = jnp.exp(s).astype(jnp.bfloat16)
        vv = jnp.concatenate([v, ones], axis=1)
        acc = jnp.dot(eb, vv, preferred_element_type=jnp.float32)
        outs.append(acc[:, :_HD] * (1.0 / acc[:, _HD:_HD + 1]))
    o_ref[...] = jnp.concatenate(outs, axis=1).astype(jnp.bfloat16)


def _att_call(q, k, v):
    grid = (_H // 2, _S // _ABLK)
    qo_spec = pl.BlockSpec((_ABLK, 2 * _HD), lambda h, i: (i, h))
    kv_spec = pl.BlockSpec((_S, 2 * _HD), lambda h, i: (0, h))
    return pl.pallas_call(
        _att_body,
        grid=grid,
        in_specs=[qo_spec, kv_spec, kv_spec],
        out_specs=qo_spec,
        out_shape=jax.ShapeDtypeStruct((_S, _DIM), jnp.bfloat16),
    )(q, k, v)


def _post_body(x_ref, o_ref, wo_ref, fnw_ref, gw_ref, h_ref, xf_ref, gs_ref):
    h = x_ref[...] + _dot_t(o_ref[...], wo_ref[...])
    xf = _rms(h, fnw_ref[...])
    h_ref[...] = h
    xf_ref[...] = xf
    gs_ref[...] = _dot_t(xf, gw_ref[...])


def _post_call(x2, o, wob, fnw, gate_w):
    n = _S // _SBLK
    blk_s = pl.BlockSpec((_SBLK, _DIM), lambda i: (i, 0))
    w_full = pl.BlockSpec((_DIM, _DIM), lambda i: (0, 0))
    return pl.pallas_call(
        _post_body,
        grid=(n,),
        in_specs=[blk_s, blk_s, w_full,
                  pl.BlockSpec((1, _DIM), lambda i: (0, 0)),
                  pl.BlockSpec((_E, _DIM), lambda i: (0, 0))],
        out_specs=[blk_s, blk_s, pl.BlockSpec((_SBLK, _E), lambda i: (i, 0))],
        out_shape=[jax.ShapeDtypeStruct((_S, _DIM), jnp.float32),
                   jax.ShapeDtypeStruct((_S, _DIM), jnp.float32),
                   jax.ShapeDtypeStruct((_S, _E), jnp.float32)],
    )(x2, o, wob, fnw, gate_w)


def _shared_body(xf_ref, sw1_ref, sw3_ref, sw2_ref, sh_ref):
    xfb = xf_ref[...].astype(jnp.bfloat16)
    g1 = _dot_t(xfb, sw1_ref[...])
    g3 = _dot_t(xfb, sw3_ref[...])
    gg = (g1 * jax.nn.sigmoid(g1) * g3).astype(jnp.bfloat16)
    sh_ref[...] = _dot_t(gg, sw2_ref[...])


def _shared_call(xf, sw1b, sw3b, sw2b):
    n = _S // _SBLK
    blk_s = pl.BlockSpec((_SBLK, _DIM), lambda i: (i, 0))
    w_full = pl.BlockSpec((_DIM, _DIM), lambda i: (0, 0))
    return pl.pallas_call(
        _shared_body,
        grid=(n,),
        in_specs=[blk_s, w_full, w_full, w_full],
        out_specs=blk_s,
        out_shape=jax.ShapeDtypeStruct((_S, _DIM), jnp.float32),
    )(xf, sw1b, sw3b, sw2b)


def _route_body(gs_ref, posi_ref, ew_ref, cnt_ref, st_ref):
    rows = _NA // 8
    gs = gs_ref[...]
    m = jnp.max(gs, axis=-1, keepdims=True)
    eg = jnp.exp(gs - m)
    probs = eg / jnp.sum(eg, axis=-1, keepdims=True)
    lane = lax.broadcasted_iota(jnp.int32, (_S, _E), 1)
    m1 = jnp.max(probs, axis=-1, keepdims=True)
    i1 = jnp.min(jnp.where(probs == m1, lane, _E), axis=-1, keepdims=True)
    oh1 = lane == i1
    probs2 = jnp.where(oh1, -jnp.inf, probs)
    m2 = jnp.max(probs2, axis=-1, keepdims=True)
    i2 = jnp.min(jnp.where(probs2 == m2, lane, _E), axis=-1, keepdims=True)
    oh2 = lane == i2

    wsum = m1 + m2 + 1e-9
    ew_ref[...] = jnp.concatenate([m1 / wsum, m2 / wsum], axis=1)

    oh = jnp.concatenate([oh1, oh2], axis=0).astype(jnp.float32)
    counts = jnp.sum(oh, axis=0, keepdims=True)
    cnt_ref[...] = counts.astype(jnp.int32)

    pc = jnp.ceil(counts / _FBLK) * _FBLK
    tri = (lax.broadcasted_iota(jnp.int32, (_E, _E), 0)
           < lax.broadcasted_iota(jnp.int32, (_E, _E), 1)).astype(jnp.float32)
    po = jnp.dot(pc, tri, preferred_element_type=jnp.float32,
                 precision=lax.Precision.HIGHEST)

    rank = jnp.concatenate([jnp.zeros((1, _E), jnp.float32), oh[:-1]], axis=0)
    d = 1
    while d < _NA:
        rank = rank + jnp.concatenate(
            [jnp.zeros((d, _E), jnp.float32), rank[:-d]], axis=0)
        d *= 2
    posb = jnp.sum(oh * (po + rank), axis=1, keepdims=True)
    posi_ref[...] = posb.astype(jnp.int32)

    cols = 512
    tokh = (lax.broadcasted_iota(jnp.int32, (2, _NA), 1) % _S
            ).astype(jnp.float32)
    lhs = jnp.where(lax.broadcasted_iota(jnp.int32, (2, _NA), 0) == 0,
                    tokh, 1.0)
    for cb in range(_NPAD // cols):
        sg = (lax.broadcasted_iota(jnp.int32, (_NA, cols), 1)
              + cb * cols).astype(jnp.float32)
        eqf = (posb == sg).astype(jnp.float32)
        red = jnp.dot(lhs, eqf, preferred_element_type=jnp.float32,
                      precision=lax.Precision.HIGHEST)
        st, hit = red[0:1], red[1:2]
        fb = (lax.broadcasted_iota(jnp.int32, (1, cols), 1) + cb * cols) % _S
        st_ref[:, cb * cols:(cb + 1) * cols] = jnp.where(
            hit > 0.0, st, fb.astype(jnp.float32)).astype(jnp.int32)


def _route_call(gs):
    return pl.pallas_call(
        _route_body,
        grid=(1,),
        in_specs=[pl.BlockSpec((_S, _E), lambda b: (0, 0))],
        out_specs=[pl.BlockSpec((_NA, 1), lambda b: (0, 0)),
                   pl.BlockSpec((_S, 2), lambda b: (0, 0)),
                   pl.BlockSpec((1, _E), lambda b: (0, 0)),
                   pl.BlockSpec((1, _NPAD), lambda b: (0, 0))],
        out_shape=[jax.ShapeDtypeStruct((_NA, 1), jnp.int32),
                   jax.ShapeDtypeStruct((_S, 2), jnp.float32),
                   jax.ShapeDtypeStruct((1, _E), jnp.int32),
                   jax.ShapeDtypeStruct((1, _NPAD), jnp.int32)],
    )(gs)


def _sc_gather(table, idx, nrows_out):
    nw = 32
    per_w = nrows_out // nw
    ch = 32
    nch = per_w // ch
    mesh = plsc.VectorSubcoreMesh(core_axis_name="c", subcore_axis_name="s")

    @functools.partial(
        pl.kernel,
        out_type=jax.ShapeDtypeStruct((nrows_out, _DIM), jnp.float32),
        mesh=mesh,
        scratch_types=[pltpu.VMEM((per_w,), jnp.int32),
                       pltpu.VMEM((ch, _DIM), jnp.float32),
                       pltpu.VMEM((ch, _DIM), jnp.float32),
                       pltpu.SemaphoreType.DMA,
                       pltpu.SemaphoreType.DMA],
    )
    def gk(table_hbm, idx_hbm, out_hbm, idx_v, rows0, rows1, sem0, sem1):
        wid = lax.axis_index("s") * 2 + lax.axis_index("c")
        base = wid * per_w
        pltpu.sync_copy(idx_hbm.at[pl.ds(base, per_w)], idx_v)
        bufs = (rows0, rows1)
        sems = (sem0, sem1)
        pend = pltpu.async_copy(table_hbm.at[idx_v.at[pl.ds(0, ch)]],
                                bufs[0], sems[0])
        for ci in range(1, nch):
            nxt = pltpu.async_copy(
                table_hbm.at[idx_v.at[pl.ds(ci * ch, ch)]],
                bufs[ci % 2], sems[ci % 2])
            pend.wait()
            pltpu.sync_copy(bufs[(ci - 1) % 2],
                            out_hbm.at[pl.ds(base + (ci - 1) * ch, ch)])
            pend = nxt
        pend.wait()
        pltpu.sync_copy(bufs[(nch - 1) % 2],
                        out_hbm.at[pl.ds(base + (nch - 1) * ch, ch)])

    return gk(table, idx)


def _ffn_body(sp_ref, xs_ref, w1_ref, w2_ref, w3_ref, eo_ref):
    b = pl.program_id(0)

    @pl.when(b < sp_ref[_NFB])
    def _():
        xsb = xs_ref[...].astype(jnp.bfloat16)
        h1 = _dot_t(xsb, w1_ref[0])
        h3 = _dot_t(xsb, w3_ref[0])
        gg = (h1 * jax.nn.sigmoid(h1) * h3).astype(jnp.bfloat16)
        eo_ref[...] = _dot_t(gg, w2_ref[0])


def _ffn_call(sp, xs, w1b, w2b, w3b):
    def _row_idx(b, sp_ref):
        return (jnp.minimum(b, sp_ref[_NFB] - 1), 0)

    grid_spec = pltpu.PrefetchScalarGridSpec(
        num_scalar_prefetch=1,
        grid=(_NFB,),
        in_specs=[
            pl.BlockSpec((_FBLK, _DIM), _row_idx),
            pl.BlockSpec((1, _HID, _DIM), lambda b, sp: (sp[b], 0, 0)),
            pl.BlockSpec((1, _DIM, _HID), lambda b, sp: (sp[b], 0, 0)),
            pl.BlockSpec((1, _HID, _DIM), lambda b, sp: (sp[b], 0, 0)),
        ],
        out_specs=pl.BlockSpec((_FBLK, _DIM), _row_idx),
    )
    return pl.pallas_call(
        _ffn_body,
        grid_spec=grid_spec,
        out_shape=jax.ShapeDtypeStruct((_NPAD, _DIM), jnp.float32),
    )(sp, xs, w1b, w2b, w3b)


def _comb_body(h_ref, sh_ref, ew_ref, y0_ref, y1_ref, out_ref):
    w = ew_ref[...]
    out_ref[...] = (h_ref[...] + sh_ref[...] + w[:, 0:1] * y0_ref[...]
                    + w[:, 1:2] * y1_ref[...])


def _comb_call(h, sh, ew, yg):
    n = _S // _SBLK
    blk_s = pl.BlockSpec((_SBLK, _DIM), lambda i: (i, 0))
    return pl.pallas_call(
        _comb_body,
        grid=(n,),
        in_specs=[blk_s, blk_s,
                  pl.BlockSpec((_SBLK, 2), lambda i: (i, 0)),
                  pl.BlockSpec((_SBLK, _DIM), lambda i: (i, 0)),
                  pl.BlockSpec((_SBLK, _DIM), lambda i: (n + i, 0))],
        out_specs=blk_s,
        out_shape=jax.ShapeDtypeStruct((_S, _DIM), jnp.float32),
    )(h, sh, ew, yg, yg)


def kernel(x, freqs_cis, att_norm_w, wq, wk, wv, wo, ffn_norm_w, gate_w,
           w1, w2, w3, sw1, sw2, sw3):
    x2 = x.reshape(_S, _DIM)
    fr, fi = freqs_cis[..., 0], freqs_cis[..., 1]
    frw = jnp.tile(jnp.repeat(fr, 2, axis=1), (1, 128 // _HD))
    fiw = jnp.tile(jnp.repeat(fi, 2, axis=1), (1, 128 // _HD))
    even = (jnp.arange(128) % 2 == 0)[None, :]
    c_tab = frw
    d1_tab = jnp.where(even, -fiw, 0.0)
    d2_tab = jnp.where(even, 0.0, fiw)

    bf = jnp.bfloat16
    q, k, v = _qkv_call(x2, c_tab, d1_tab, d2_tab, att_norm_w.reshape(1, _DIM),
                        wq.astype(bf), wk.astype(bf), wv.astype(bf))
    o = _att_call(q, k, v)
    h, xf, gs = _post_call(x2, o, wo.astype(bf),
                           ffn_norm_w.reshape(1, _DIM), gate_w)
    posi, ew, cnt, st = _route_call(gs)
    st_i = st.reshape(_NPAD)

    counts = cnt.reshape(_E)
    pc = ((counts + _FBLK - 1) // _FBLK) * _FBLK
    po = jnp.concatenate([jnp.zeros((1,), jnp.int32), jnp.cumsum(pc)[:-1]])
    ends = po + pc
    nblk = jnp.sum(pc) // _FBLK
    bidx = jnp.arange(_NFB, dtype=jnp.int32)
    beid_raw = jnp.minimum(
        jnp.sum((bidx[:, None] * _FBLK >= ends[None, :]).astype(jnp.int32),
                axis=1), _E - 1)
    fill = beid_raw[nblk - 1]
    beid = jnp.where(bidx < nblk, beid_raw, fill)
    sp = jnp.concatenate([beid, nblk[None]]).astype(jnp.int32)

    xs = _sc_gather(xf, st_i, _NPAD)
    sh = _shared_call(xf, sw1.astype(bf), sw3.astype(bf), sw2.astype(bf))
    eo = _ffn_call(sp, xs, w1.astype(bf), w2.astype(bf), w3.astype(bf))
    yg = _sc_gather(eo, posi.reshape(_NA), _NA)
    out = _comb_call(h, sh, ew, yg)
    return out.reshape(_B, _S, _DIM)


# speedup vs baseline: 1.0914x; 1.0914x over previous
import functools
import math

import jax
import jax.numpy as jnp
from jax import lax
from jax.experimental import pallas as pl
from jax.experimental.pallas import tpu as pltpu
from jax.experimental.pallas import tpu_sc as plsc

_B, _S, _DIM, _H, _HD = 1, 2048, 1024, 16, 64
_E, _TOPK, _HID, _SHID = 8, 2, 1024, 1024
_EPS = 1e-6

_SBLK = 256
_ABLK = 512
_FBLK = 128
_NA = _TOPK * _S
_NPAD = _NA + _E * _FBLK
_NFB = _NPAD // _FBLK


def _rms(x, w):
    return x * lax.rsqrt(jnp.mean(x * x, axis=-1, keepdims=True) + _EPS) * w


def _dot_t(a, b):
    return lax.dot_general(a, b, (((1,), (1,)), ((), ())),
                           preferred_element_type=jnp.float32)


def _rope(y, cc, d1, d2):
    left = jnp.concatenate([y[:, 1:], y[:, :1]], axis=1)
    right = jnp.concatenate([y[:, -1:], y[:, :-1]], axis=1)
    return y * cc + left * d1 + right * d2


def _qkv_body(x_ref, c_ref, d1_ref, d2_ref, anw_ref, wq_ref, wk_ref, wv_ref,
              q_ref, k_ref, v_ref):
    xn = _rms(x_ref[...], anw_ref[...]).astype(jnp.bfloat16)
    cc = jnp.concatenate([c_ref[...]] * (_DIM // 128), axis=1)
    d1 = jnp.concatenate([d1_ref[...]] * (_DIM // 128), axis=1)
    d2 = jnp.concatenate([d2_ref[...]] * (_DIM // 128), axis=1)
    q = _dot_t(xn, wq_ref[...])
    k = _dot_t(xn, wk_ref[...])
    q_ref[...] = _rope(q, cc, d1, d2).astype(jnp.bfloat16)
    k_ref[...] = _rope(k, cc, d1, d2).astype(jnp.bfloat16)
    v_ref[...] = _dot_t(xn, wv_ref[...]).astype(jnp.bfloat16)


def _qkv_call(x2, c_tab, d1_tab, d2_tab, anw, wqb, wkb, wvb):
    n = _S // _SBLK
    blk_s = pl.BlockSpec((_SBLK, _DIM), lambda i: (i, 0))
    blk_t = pl.BlockSpec((_SBLK, 128), lambda i: (i, 0))
    w_full = pl.BlockSpec((_DIM, _DIM), lambda i: (0, 0))
    return pl.pallas_call(
        _qkv_body,
        grid=(n,),
        in_specs=[blk_s, blk_t, blk_t, blk_t,
                  pl.BlockSpec((1, _DIM), lambda i: (0, 0)),
                  w_full, w_full, w_full],
        out_specs=[blk_s, blk_s, blk_s],
        out_shape=[jax.ShapeDtypeStruct((_S, _DIM), jnp.bfloat16)] * 3,
    )(x2, c_tab, d1_tab, d2_tab, anw, wqb, wkb, wvb)


def _att_body(q_ref, k_ref, v_ref, o_ref):
    ones = jnp.ones((_S, 1), jnp.bfloat16)
    outs = []
    for p in range(2):
        q = q_ref[:, p * _HD:(p + 1) * _HD]
        k = k_ref[:, p * _HD:(p + 1) * _HD]
        v = v_ref[:, p * _HD:(p + 1) * _HD]
        s = _dot_t(q, k) * (1.0 / math.sqrt(_HD))
        eb = jnp.exp(s).astype(jnp.bfloat16)
        vv = jnp.concatenate([v, ones], axis=1)
        acc = jnp.dot(eb, vv, preferred_element_type=jnp.float32)
        outs.append(acc[:, :_HD] * (1.0 / acc[:, _HD:_HD + 1]))
    o_ref[...] = jnp.concatenate(outs, axis=1).astype(jnp.bfloat16)


def _att_call(q, k, v):
    grid = (_H // 2, _S // _ABLK)
    qo_spec = pl.BlockSpec((_ABLK, 2 * _HD), lambda h, i: (i, h))
    kv_spec = pl.BlockSpec((_S, 2 * _HD), lambda h, i: (0, h))
    return pl.pallas_call(
        _att_body,
        grid=grid,
        in_specs=[qo_spec, kv_spec, kv_spec],
        out_specs=qo_spec,
        out_shape=jax.ShapeDtypeStruct((_S, _DIM), jnp.bfloat16),
    )(q, k, v)


def _post_body(x_ref, o_ref, wo_ref, fnw_ref, gw_ref, h_ref, xf_ref, gs_ref):
    h = x_ref[...] + _dot_t(o_ref[...], wo_ref[...])
    xf = _rms(h, fnw_ref[...])
    h_ref[...] = h
    xf_ref[...] = xf
    gs_ref[...] = _dot_t(xf, gw_ref[...])


def _post_call(x2, o, wob, fnw, gate_w):
    n = _S // _SBLK
    blk_s = pl.BlockSpec((_SBLK, _DIM), lambda i: (i, 0))
    w_full = pl.BlockSpec((_DIM, _DIM), lambda i: (0, 0))
    return pl.pallas_call(
        _post_body,
        grid=(n,),
        in_specs=[blk_s, blk_s, w_full,
                  pl.BlockSpec((1, _DIM), lambda i: (0, 0)),
                  pl.BlockSpec((_E, _DIM), lambda i: (0, 0))],
        out_specs=[blk_s, blk_s, pl.BlockSpec((_SBLK, _E), lambda i: (i, 0))],
        out_shape=[jax.ShapeDtypeStruct((_S, _DIM), jnp.float32),
                   jax.ShapeDtypeStruct((_S, _DIM), jnp.float32),
                   jax.ShapeDtypeStruct((_S, _E), jnp.float32)],
    )(x2, o, wob, fnw, gate_w)


def _shared_body(xf_ref, sw1_ref, sw3_ref, sw2_ref, sh_ref):
    xfb = xf_ref[...].astype(jnp.bfloat16)
    g1 = _dot_t(xfb, sw1_ref[...])
    g3 = _dot_t(xfb, sw3_ref[...])
    gg = (g1 * jax.nn.sigmoid(g1) * g3).astype(jnp.bfloat16)
    sh_ref[...] = _dot_t(gg, sw2_ref[...])


def _shared_call(xf, sw1b, sw3b, sw2b):
    n = _S // _SBLK
    blk_s = pl.BlockSpec((_SBLK, _DIM), lambda i: (i, 0))
    w_full = pl.BlockSpec((_DIM, _DIM), lambda i: (0, 0))
    return pl.pallas_call(
        _shared_body,
        grid=(n,),
        in_specs=[blk_s, w_full, w_full, w_full],
        out_specs=blk_s,
        out_shape=jax.ShapeDtypeStruct((_S, _DIM), jnp.float32),
    )(xf, sw1b, sw3b, sw2b)


def _route_body(gs_ref, posi_ref, ew_ref, cnt_ref, st_ref):
    rows = _NA // 8
    gs = gs_ref[...]
    m = jnp.max(gs, axis=-1, keepdims=True)
    eg = jnp.exp(gs - m)
    probs = eg / jnp.sum(eg, axis=-1, keepdims=True)
    lane = lax.broadcasted_iota(jnp.int32, (_S, _E), 1)
    m1 = jnp.max(probs, axis=-1, keepdims=True)
    i1 = jnp.min(jnp.where(probs == m1, lane, _E), axis=-1, keepdims=True)
    oh1 = lane == i1
    probs2 = jnp.where(oh1, -jnp.inf, probs)
    m2 = jnp.max(probs2, axis=-1, keepdims=True)
    i2 = jnp.min(jnp.where(probs2 == m2, lane, _E), axis=-1, keepdims=True)
    oh2 = lane == i2

    wsum = m1 + m2 + 1e-9
    ew_ref[...] = jnp.concatenate([m1 / wsum, m2 / wsum], axis=1)

    oh = jnp.concatenate([oh1, oh2], axis=0).astype(jnp.float32)
    counts = jnp.sum(oh, axis=0, keepdims=True)
    cnt_ref[...] = counts.astype(jnp.int32)

    pc = jnp.ceil(counts / _FBLK) * _FBLK
    tri = (lax.broadcasted_iota(jnp.int32, (_E, _E), 0)
           < lax.broadcasted_iota(jnp.int32, (_E, _E), 1)).astype(jnp.float32)
    po = jnp.dot(pc, tri, preferred_element_type=jnp.float32)

    rank = jnp.concatenate([jnp.zeros((1, _E), jnp.float32), oh[:-1]], axis=0)
    d = 1
    while d < _NA:
        rank = rank + jnp.concatenate(
            [jnp.zeros((d, _E), jnp.float32), rank[:-d]], axis=0)
        d *= 2
    posb = jnp.sum(oh * (po + rank), axis=1, keepdims=True)
    posi_ref[...] = posb.astype(jnp.int32)

    cols = 512
    tok = lax.broadcasted_iota(jnp.int32, (3, _NA), 1) % _S
    lrow = lax.broadcasted_iota(jnp.int32, (3, _NA), 0)
    lhs = jnp.where(lrow == 0, tok // 128,
                    jnp.where(lrow == 1, tok % 128, 1)
                    ).astype(jnp.bfloat16)
    for cb in range(_NPAD // cols):
        sg = (lax.broadcasted_iota(jnp.int32, (_NA, cols), 1)
              + cb * cols).astype(jnp.float32)
        eqb = (posb == sg).astype(jnp.bfloat16)
        red = jnp.dot(lhs, eqb, preferred_element_type=jnp.float32)
        st = red[0:1] * 128.0 + red[1:2]
        hit = red[2:3]
        fb = (lax.broadcasted_iota(jnp.int32, (1, cols), 1) + cb * cols) % _S
        st_ref[:, cb * cols:(cb + 1) * cols] = jnp.where(
            hit > 0.0, st, fb.astype(jnp.float32)).astype(jnp.int32)


def _route_call(gs):
    return pl.pallas_call(
        _route_body,
        grid=(1,),
        in_specs=[pl.BlockSpec((_S, _E), lambda b: (0, 0))],
        out_specs=[pl.BlockSpec((_NA, 1), lambda b: (0, 0)),
                   pl.BlockSpec((_S, 2), lambda b: (0, 0)),
                   pl.BlockSpec((1, _E), lambda b: (0, 0)),
                   pl.BlockSpec((1, _NPAD), lambda b: (0, 0))],
        out_shape=[jax.ShapeDtypeStruct((_NA, 1), jnp.int32),
                   jax.ShapeDtypeStruct((_S, 2), jnp.float32),
                   jax.ShapeDtypeStruct((1, _E), jnp.int32),
                   jax.ShapeDtypeStruct((1, _NPAD), jnp.int32)],
    )(gs)


def _sc_gather(table, idx, nrows_out):
    nw = 32
    per_w = nrows_out // nw
    ch = 32
    nch = per_w // ch
    mesh = plsc.VectorSubcoreMesh(core_axis_name="c", subcore_axis_name="s")

    @functools.partial(
        pl.kernel,
        out_type=jax.ShapeDtypeStruct((nrows_out, _DIM), jnp.float32),
        mesh=mesh,
        scratch_types=[pltpu.VMEM((per_w,), jnp.int32),
                       pltpu.VMEM((ch, _DIM), jnp.float32),
                       pltpu.VMEM((ch, _DIM), jnp.float32),
                       pltpu.SemaphoreType.DMA,
                       pltpu.SemaphoreType.DMA],
    )
    def gk(table_hbm, idx_hbm, out_hbm, idx_v, rows0, rows1, sem0, sem1):
        wid = lax.axis_index("s") * 2 + lax.axis_index("c")
        base = wid * per_w
        pltpu.sync_copy(idx_hbm.at[pl.ds(base, per_w)], idx_v)
        bufs = (rows0, rows1)
        sems = (sem0, sem1)
        pend = pltpu.async_copy(table_hbm.at[idx_v.at[pl.ds(0, ch)]],
                                bufs[0], sems[0])
        for ci in range(1, nch):
            nxt = pltpu.async_copy(
                table_hbm.at[idx_v.at[pl.ds(ci * ch, ch)]],
                bufs[ci % 2], sems[ci % 2])
            pend.wait()
            pltpu.sync_copy(bufs[(ci - 1) % 2],
                            out_hbm.at[pl.ds(base + (ci - 1) * ch, ch)])
            pend = nxt
        pend.wait()
        pltpu.sync_copy(bufs[(nch - 1) % 2],
                        out_hbm.at[pl.ds(base + (nch - 1) * ch, ch)])

    return gk(table, idx)


def _ffn_body(sp_ref, xs_ref, w1_ref, w2_ref, w3_ref, eo_ref):
    b = pl.program_id(0)

    @pl.when(b < sp_ref[_NFB])
    def _():
        xsb = xs_ref[...].astype(jnp.bfloat16)
        h1 = _dot_t(xsb, w1_ref[0])
        h3 = _dot_t(xsb, w3_ref[0])
        gg = (h1 * jax.nn.sigmoid(h1) * h3).astype(jnp.bfloat16)
        eo_ref[...] = _dot_t(gg, w2_ref[0])


def _ffn_call(sp, xs, w1b, w2b, w3b):
    def _row_idx(b, sp_ref):
        return (jnp.minimum(b, sp_ref[_NFB] - 1), 0)

    grid_spec = pltpu.PrefetchScalarGridSpec(
        num_scalar_prefetch=1,
        grid=(_NFB,),
        in_specs=[
            pl.BlockSpec((_FBLK, _DIM), _row_idx),
            pl.BlockSpec((1, _HID, _DIM), lambda b, sp: (sp[b], 0, 0)),
            pl.BlockSpec((1, _DIM, _HID), lambda b, sp: (sp[b], 0, 0)),
            pl.BlockSpec((1, _HID, _DIM), lambda b, sp: (sp[b], 0, 0)),
        ],
        out_specs=pl.BlockSpec((_FBLK, _DIM), _row_idx),
    )
    return pl.pallas_call(
        _ffn_body,
        grid_spec=grid_spec,
        out_shape=jax.ShapeDtypeStruct((_NPAD, _DIM), jnp.float32),
    )(sp, xs, w1b, w2b, w3b)


def _comb_body(h_ref, sh_ref, ew_ref, y0_ref, y1_ref, out_ref):
    w = ew_ref[...]
    out_ref[...] = (h_ref[...] + sh_ref[...] + w[:, 0:1] * y0_ref[...]
                    + w[:, 1:2] * y1_ref[...])


def _comb_call(h, sh, ew, yg):
    n = _S // _SBLK
    blk_s = pl.BlockSpec((_SBLK, _DIM), lambda i: (i, 0))
    return pl.pallas_call(
        _comb_body,
        grid=(n,),
        in_specs=[blk_s, blk_s,
                  pl.BlockSpec((_SBLK, 2), lambda i: (i, 0)),
                  pl.BlockSpec((_SBLK, _DIM), lambda i: (i, 0)),
                  pl.BlockSpec((_SBLK, _DIM), lambda i: (n + i, 0))],
        out_specs=blk_s,
        out_shape=jax.ShapeDtypeStruct((_S, _DIM), jnp.float32),
    )(h, sh, ew, yg, yg)


def kernel(x, freqs_cis, att_norm_w, wq, wk, wv, wo, ffn_norm_w, gate_w,
           w1, w2, w3, sw1, sw2, sw3):
    x2 = x.reshape(_S, _DIM)
    fr, fi = freqs_cis[..., 0], freqs_cis[..., 1]
    frw = jnp.tile(jnp.repeat(fr, 2, axis=1), (1, 128 // _HD))
    fiw = jnp.tile(jnp.repeat(fi, 2, axis=1), (1, 128 // _HD))
    even = (jnp.arange(128) % 2 == 0)[None, :]
    c_tab = frw
    d1_tab = jnp.where(even, -fiw, 0.0)
    d2_tab = jnp.where(even, 0.0, fiw)

    bf = jnp.bfloat16
    q, k, v = _qkv_call(x2, c_tab, d1_tab, d2_tab, att_norm_w.reshape(1, _DIM),
                        wq.astype(bf), wk.astype(bf), wv.astype(bf))
    o = _att_call(q, k, v)
    h, xf, gs = _post_call(x2, o, wo.astype(bf),
                           ffn_norm_w.reshape(1, _DIM), gate_w)
    posi, ew, cnt, st = _route_call(gs)
    st_i = st.reshape(_NPAD)

    counts = cnt.reshape(_E)
    pc = ((counts + _FBLK - 1) // _FBLK) * _FBLK
    po = jnp.concatenate([jnp.zeros((1,), jnp.int32), jnp.cumsum(pc)[:-1]])
    ends = po + pc
    nblk = jnp.sum(pc) // _FBLK
    bidx = jnp.arange(_NFB, dtype=jnp.int32)
    beid_raw = jnp.minimum(
        jnp.sum((bidx[:, None] * _FBLK >= ends[None, :]).astype(jnp.int32),
                axis=1), _E - 1)
    fill = beid_raw[nblk - 1]
    beid = jnp.where(bidx < nblk, beid_raw, fill)
    sp = jnp.concatenate([beid, nblk[None]]).astype(jnp.int32)

    xs = _sc_gather(xf, st_i, _NPAD)
    sh = _shared_call(xf, sw1.astype(bf), sw3.astype(bf), sw2.astype(bf))
    eo = _ffn_call(sp, xs, w1.astype(bf), w2.astype(bf), w3.astype(bf))
    yg = _sc_gather(eo, posi.reshape(_NA), _NA)
    out = _comb_call(h, sh, ew, yg)
    return out.reshape(_B, _S, _DIM)


# speedup vs baseline: 1.1127x; 1.0195x over previous
import functools
import math

import jax
import jax.numpy as jnp
from jax import lax
from jax.experimental import pallas as pl
from jax.experimental.pallas import tpu as pltpu
from jax.experimental.pallas import tpu_sc as plsc

_B, _S, _DIM, _H, _HD = 1, 2048, 1024, 16, 64
_E, _TOPK, _HID, _SHID = 8, 2, 1024, 1024
_EPS = 1e-6

_SBLK = 256
_ABLK = 1024
_FBLK = 128
_NA = _TOPK * _S
_NPAD = _NA + _E * _FBLK
_NFB = _NPAD // _FBLK


def _rms(x, w):
    return x * lax.rsqrt(jnp.mean(x * x, axis=-1, keepdims=True) + _EPS) * w


def _dot_t(a, b):
    return lax.dot_general(a, b, (((1,), (1,)), ((), ())),
                           preferred_element_type=jnp.float32)


def _rope(y, cc, d1, d2):
    left = jnp.concatenate([y[:, 1:], y[:, :1]], axis=1)
    right = jnp.concatenate([y[:, -1:], y[:, :-1]], axis=1)
    return y * cc + left * d1 + right * d2


def _qkv_body(x_ref, c_ref, d1_ref, d2_ref, anw_ref, wq_ref, wk_ref, wv_ref,
              q_ref, k_ref, v_ref):
    xn = _rms(x_ref[...], anw_ref[...]).astype(jnp.bfloat16)
    cc = jnp.concatenate([c_ref[...]] * (_DIM // 128), axis=1)
    d1 = jnp.concatenate([d1_ref[...]] * (_DIM // 128), axis=1)
    d2 = jnp.concatenate([d2_ref[...]] * (_DIM // 128), axis=1)
    q = _dot_t(xn, wq_ref[...])
    k = _dot_t(xn, wk_ref[...])
    q_ref[...] = _rope(q, cc, d1, d2).astype(jnp.bfloat16)
    k_ref[...] = _rope(k, cc, d1, d2).astype(jnp.bfloat16)
    v_ref[...] = _dot_t(xn, wv_ref[...]).astype(jnp.bfloat16)


def _qkv_call(x2, c_tab, d1_tab, d2_tab, anw, wqb, wkb, wvb):
    n = _S // _SBLK
    blk_s = pl.BlockSpec((_SBLK, _DIM), lambda i: (i, 0))
    blk_t = pl.BlockSpec((_SBLK, 128), lambda i: (i, 0))
    w_full = pl.BlockSpec((_DIM, _DIM), lambda i: (0, 0))
    return pl.pallas_call(
        _qkv_body,
        grid=(n,),
        in_specs=[blk_s, blk_t, blk_t, blk_t,
                  pl.BlockSpec((1, _DIM), lambda i: (0, 0)),
                  w_full, w_full, w_full],
        out_specs=[blk_s, blk_s, blk_s],
        out_shape=[jax.ShapeDtypeStruct((_S, _DIM), jnp.bfloat16)] * 3,
    )(x2, c_tab, d1_tab, d2_tab, anw, wqb, wkb, wvb)


def _att_body(q_ref, k_ref, v_ref, o_ref):
    ones = jnp.ones((_S, 1), jnp.bfloat16)
    outs = []
    for p in range(2):
        q = q_ref[:, p * _HD:(p + 1) * _HD]
        k = k_ref[:, p * _HD:(p + 1) * _HD]
        v = v_ref[:, p * _HD:(p + 1) * _HD]
        s = _dot_t(q, k) * (1.0 / math.sqrt(_HD))
        eb = jnp.exp(s).astype(jnp.bfloat16)
        vv = jnp.concatenate([v, ones], axis=1)
        acc = jnp.dot(eb, vv, preferred_element_type=jnp.float32)
        outs.append(acc[:, :_HD] * (1.0 / acc[:, _HD:_HD + 1]))
    o_ref[...] = jnp.concatenate(outs, axis=1).astype(jnp.bfloat16)


def _att_call(q, k, v):
    grid = (_H // 2, _S // _ABLK)
    qo_spec = pl.BlockSpec((_ABLK, 2 * _HD), lambda h, i: (i, h))
    kv_spec = pl.BlockSpec((_S, 2 * _HD), lambda h, i: (0, h))
    return pl.pallas_call(
        _att_body,
        grid=grid,
        in_specs=[qo_spec, kv_spec, kv_spec],
        out_specs=qo_spec,
        out_shape=jax.ShapeDtypeStruct((_S, _DIM), jnp.bfloat16),
    )(q, k, v)


def _post_body(x_ref, o_ref, wo_ref, fnw_ref, gw_ref, h_ref, xf_ref, gs_ref):
    h = x_ref[...] + _dot_t(o_ref[...], wo_ref[...])
    xf = _rms(h, fnw_ref[...])
    h_ref[...] = h
    xf_ref[...] = xf
    gs_ref[...] = _dot_t(xf, gw_ref[...])


def _post_call(x2, o, wob, fnw, gate_w):
    n = _S // _SBLK
    blk_s = pl.BlockSpec((_SBLK, _DIM), lambda i: (i, 0))
    w_full = pl.BlockSpec((_DIM, _DIM), lambda i: (0, 0))
    return pl.pallas_call(
        _post_body,
        grid=(n,),
        in_specs=[blk_s, blk_s, w_full,
                  pl.BlockSpec((1, _DIM), lambda i: (0, 0)),
                  pl.BlockSpec((_E, _DIM), lambda i: (0, 0))],
        out_specs=[blk_s, blk_s, pl.BlockSpec((_SBLK, _E), lambda i: (i, 0))],
        out_shape=[jax.ShapeDtypeStruct((_S, _DIM), jnp.float32),
                   jax.ShapeDtypeStruct((_S, _DIM), jnp.float32),
                   jax.ShapeDtypeStruct((_S, _E), jnp.float32)],
    )(x2, o, wob, fnw, gate_w)


def _shared_body(xf_ref, sw1_ref, sw3_ref, sw2_ref, sh_ref):
    xfb = xf_ref[...].astype(jnp.bfloat16)
    g1 = _dot_t(xfb, sw1_ref[...])
    g3 = _dot_t(xfb, sw3_ref[...])
    gg = (g1 * jax.nn.sigmoid(g1) * g3).astype(jnp.bfloat16)
    sh_ref[...] = _dot_t(gg, sw2_ref[...])


def _shared_call(xf, sw1b, sw3b, sw2b):
    n = _S // _SBLK
    blk_s = pl.BlockSpec((_SBLK, _DIM), lambda i: (i, 0))
    w_full = pl.BlockSpec((_DIM, _DIM), lambda i: (0, 0))
    return pl.pallas_call(
        _shared_body,
        grid=(n,),
        in_specs=[blk_s, w_full, w_full, w_full],
        out_specs=blk_s,
        out_shape=jax.ShapeDtypeStruct((_S, _DIM), jnp.float32),
    )(xf, sw1b, sw3b, sw2b)


def _route_body(gs_ref, posi_ref, ew_ref, cnt_ref, st_ref):
    rows = _NA // 8
    gs = gs_ref[...]
    m = jnp.max(gs, axis=-1, keepdims=True)
    eg = jnp.exp(gs - m)
    probs = eg / jnp.sum(eg, axis=-1, keepdims=True)
    lane = lax.broadcasted_iota(jnp.int32, (_S, _E), 1)
    m1 = jnp.max(probs, axis=-1, keepdims=True)
    i1 = jnp.min(jnp.where(probs == m1, lane, _E), axis=-1, keepdims=True)
    oh1 = lane == i1
    probs2 = jnp.where(oh1, -jnp.inf, probs)
    m2 = jnp.max(probs2, axis=-1, keepdims=True)
    i2 = jnp.min(jnp.where(probs2 == m2, lane, _E), axis=-1, keepdims=True)
    oh2 = lane == i2

    wsum = m1 + m2 + 1e-9
    ew_ref[...] = jnp.concatenate([m1 / wsum, m2 / wsum], axis=1)

    oh = jnp.concatenate([oh1, oh2], axis=0).astype(jnp.float32)
    counts = jnp.sum(oh, axis=0, keepdims=True)
    cnt_ref[...] = counts.astype(jnp.int32)

    pc = jnp.ceil(counts / _FBLK) * _FBLK
    tri = (lax.broadcasted_iota(jnp.int32, (_E, _E), 0)
           < lax.broadcasted_iota(jnp.int32, (_E, _E), 1)).astype(jnp.float32)
    po = jnp.dot(pc, tri, preferred_element_type=jnp.float32)

    rank = jnp.concatenate([jnp.zeros((1, _E), jnp.float32), oh[:-1]], axis=0)
    d = 1
    while d < _NA:
        rank = rank + jnp.concatenate(
            [jnp.zeros((d, _E), jnp.float32), rank[:-d]], axis=0)
        d *= 2
    posb = jnp.sum(oh * (po + rank), axis=1, keepdims=True)
    posi_ref[...] = posb.astype(jnp.int32)

    cols = 512
    tok = lax.broadcasted_iota(jnp.int32, (3, _NA), 1) % _S
    lrow = lax.broadcasted_iota(jnp.int32, (3, _NA), 0)
    lhs = jnp.where(lrow == 0, tok // 128,
                    jnp.where(lrow == 1, tok % 128, 1)
                    ).astype(jnp.bfloat16)
    for cb in range(_NPAD // cols):
        sg = (lax.broadcasted_iota(jnp.int32, (_NA, cols), 1)
              + cb * cols).astype(jnp.float32)
        eqb = (posb == sg).astype(jnp.bfloat16)
        red = jnp.dot(lhs, eqb, preferred_element_type=jnp.float32)
        st = red[0:1] * 128.0 + red[1:2]
        hit = red[2:3]
        fb = (lax.broadcasted_iota(jnp.int32, (1, cols), 1) + cb * cols) % _S
        st_ref[:, cb * cols:(cb + 1) * cols] = jnp.where(
            hit > 0.0, st, fb.astype(jnp.float32)).astype(jnp.int32)


def _route_call(gs):
    return pl.pallas_call(
        _route_body,
        grid=(1,),
        in_specs=[pl.BlockSpec((_S, _E), lambda b: (0, 0))],
        out_specs=[pl.BlockSpec((_NA, 1), lambda b: (0, 0)),
                   pl.BlockSpec((_S, 2), lambda b: (0, 0)),
                   pl.BlockSpec((1, _E), lambda b: (0, 0)),
                   pl.BlockSpec((1, _NPAD), lambda b: (0, 0))],
        out_shape=[jax.ShapeDtypeStruct((_NA, 1), jnp.int32),
                   jax.ShapeDtypeStruct((_S, 2), jnp.float32),
                   jax.ShapeDtypeStruct((1, _E), jnp.int32),
                   jax.ShapeDtypeStruct((1, _NPAD), jnp.int32)],
    )(gs)


def _sc_gather(table, idx, nrows_out):
    nw = 32
    per_w = nrows_out // nw
    ch = 32
    nch = per_w // ch
    mesh = plsc.VectorSubcoreMesh(core_axis_name="c", subcore_axis_name="s")

    @functools.partial(
        pl.kernel,
        out_type=jax.ShapeDtypeStruct((nrows_out, _DIM), jnp.float32),
        mesh=mesh,
        scratch_types=[pltpu.VMEM((per_w,), jnp.int32),
                       pltpu.VMEM((ch, _DIM), jnp.float32),
                       pltpu.VMEM((ch, _DIM), jnp.float32),
                       pltpu.SemaphoreType.DMA,
                       pltpu.SemaphoreType.DMA],
    )
    def gk(table_hbm, idx_hbm, out_hbm, idx_v, rows0, rows1, sem0, sem1):
        wid = lax.axis_index("s") * 2 + lax.axis_index("c")
        base = wid * per_w
        pltpu.sync_copy(idx_hbm.at[pl.ds(base, per_w)], idx_v)
        bufs = (rows0, rows1)
        sems = (sem0, sem1)
        pend = pltpu.async_copy(table_hbm.at[idx_v.at[pl.ds(0, ch)]],
                                bufs[0], sems[0])
        for ci in range(1, nch):
            nxt = pltpu.async_copy(
                table_hbm.at[idx_v.at[pl.ds(ci * ch, ch)]],
                bufs[ci % 2], sems[ci % 2])
            pend.wait()
            pltpu.sync_copy(bufs[(ci - 1) % 2],
                            out_hbm.at[pl.ds(base + (ci - 1) * ch, ch)])
            pend = nxt
        pend.wait()
        pltpu.sync_copy(bufs[(nch - 1) % 2],
                        out_hbm.at[pl.ds(base + (nch - 1) * ch, ch)])

    return gk(table, idx)


def _ffn_body(sp_ref, xs_ref, w1_ref, w2_ref, w3_ref, eo_ref):
    b = pl.program_id(0)

    @pl.when(b < sp_ref[_NFB])
    def _():
        xsb = xs_ref[...].astype(jnp.bfloat16)
        h1 = _dot_t(xsb, w1_ref[0])
        h3 = _dot_t(xsb, w3_ref[0])
        gg = (h1 * jax.nn.sigmoid(h1) * h3).astype(jnp.bfloat16)
        eo_ref[...] = _dot_t(gg, w2_ref[0])


def _ffn_call(sp, xs, w1b, w2b, w3b):
    def _row_idx(b, sp_ref):
        return (jnp.minimum(b, sp_ref[_NFB] - 1), 0)

    grid_spec = pltpu.PrefetchScalarGridSpec(
        num_scalar_prefetch=1,
        grid=(_NFB,),
        in_specs=[
            pl.BlockSpec((_FBLK, _DIM), _row_idx),
            pl.BlockSpec((1, _HID, _DIM), lambda b, sp: (sp[b], 0, 0)),
            pl.BlockSpec((1, _DIM, _HID), lambda b, sp: (sp[b], 0, 0)),
            pl.BlockSpec((1, _HID, _DIM), lambda b, sp: (sp[b], 0, 0)),
        ],
        out_specs=pl.BlockSpec((_FBLK, _DIM), _row_idx),
    )
    return pl.pallas_call(
        _ffn_body,
        grid_spec=grid_spec,
        out_shape=jax.ShapeDtypeStruct((_NPAD, _DIM), jnp.float32),
    )(sp, xs, w1b, w2b, w3b)


def _comb_body(h_ref, sh_ref, ew_ref, y0_ref, y1_ref, out_ref):
    w = ew_ref[...]
    out_ref[...] = (h_ref[...] + sh_ref[...] + w[:, 0:1] * y0_ref[...]
                    + w[:, 1:2] * y1_ref[...])


def _comb_call(h, sh, ew, yg):
    n = _S // _SBLK
    blk_s = pl.BlockSpec((_SBLK, _DIM), lambda i: (i, 0))
    return pl.pallas_call(
        _comb_body,
        grid=(n,),
        in_specs=[blk_s, blk_s,
                  pl.BlockSpec((_SBLK, 2), lambda i: (i, 0)),
                  pl.BlockSpec((_SBLK, _DIM), lambda i: (i, 0)),
                  pl.BlockSpec((_SBLK, _DIM), lambda i: (n + i, 0))],
        out_specs=blk_s,
        out_shape=jax.ShapeDtypeStruct((_S, _DIM), jnp.float32),
    )(h, sh, ew, yg, yg)


def kernel(x, freqs_cis, att_norm_w, wq, wk, wv, wo, ffn_norm_w, gate_w,
           w1, w2, w3, sw1, sw2, sw3):
    x2 = x.reshape(_S, _DIM)
    fr, fi = freqs_cis[..., 0], freqs_cis[..., 1]
    frw = jnp.tile(jnp.repeat(fr, 2, axis=1), (1, 128 // _HD))
    fiw = jnp.tile(jnp.repeat(fi, 2, axis=1), (1, 128 // _HD))
    even = (jnp.arange(128) % 2 == 0)[None, :]
    c_tab = frw
    d1_tab = jnp.where(even, -fiw, 0.0)
    d2_tab = jnp.where(even, 0.0, fiw)

    bf = jnp.bfloat16
    q, k, v = _qkv_call(x2, c_tab, d1_tab, d2_tab, att_norm_w.reshape(1, _DIM),
                        wq.astype(bf), wk.astype(bf), wv.astype(bf))
    o = _att_call(q, k, v)
    h, xf, gs = _post_call(x2, o, wo.astype(bf),
                           ffn_norm_w.reshape(1, _DIM), gate_w)
    posi, ew, cnt, st = _route_call(gs)
    st_i = st.reshape(_NPAD)

    counts = cnt.reshape(_E)
    pc = ((counts + _FBLK - 1) // _FBLK) * _FBLK
    po = jnp.concatenate([jnp.zeros((1,), jnp.int32), jnp.cumsum(pc)[:-1]])
    ends = po + pc
    nblk = jnp.sum(pc) // _FBLK
    bidx = jnp.arange(_NFB, dtype=jnp.int32)
    beid_raw = jnp.minimum(
        jnp.sum((bidx[:, None] * _FBLK >= ends[None, :]).astype(jnp.int32),
                axis=1), _E - 1)
    fill = beid_raw[nblk - 1]
    beid = jnp.where(bidx < nblk, beid_raw, fill)
    sp = jnp.concatenate([beid, nblk[None]]).astype(jnp.int32)

    xs = _sc_gather(xf, st_i, _NPAD)
    sh = _shared_call(xf, sw1.astype(bf), sw3.astype(bf), sw2.astype(bf))
    eo = _ffn_call(sp, xs, w1.astype(bf), w2.astype(bf), w3.astype(bf))
    yg = _sc_gather(eo, posi.reshape(_NA), _NA)
    out = _comb_call(h, sh, ew, yg)
    return out.reshape(_B, _S, _DIM)


# speedup vs baseline: 1.1259x; 1.0119x over previous
import functools
import math

import jax
import jax.numpy as jnp
from jax import lax
from jax.experimental import pallas as pl
from jax.experimental.pallas import tpu as pltpu
from jax.experimental.pallas import tpu_sc as plsc

_B, _S, _DIM, _H, _HD = 1, 2048, 1024, 16, 64
_E, _TOPK, _HID, _SHID = 8, 2, 1024, 1024
_EPS = 1e-6

_SBLK = 512
_ABLK = 1024
_FBLK = 128
_NA = _TOPK * _S
_NPAD = _NA + _E * _FBLK
_NFB = _NPAD // _FBLK


def _rms(x, w):
    return x * lax.rsqrt(jnp.mean(x * x, axis=-1, keepdims=True) + _EPS) * w


def _dot_t(a, b):
    return lax.dot_general(a, b, (((1,), (1,)), ((), ())),
                           preferred_element_type=jnp.float32)


def _rope(y, cc, d1, d2):
    left = jnp.concatenate([y[:, 1:], y[:, :1]], axis=1)
    right = jnp.concatenate([y[:, -1:], y[:, :-1]], axis=1)
    return y * cc + left * d1 + right * d2


def _qkv_body(x_ref, c_ref, d1_ref, d2_ref, anw_ref, wq_ref, wk_ref, wv_ref,
              q_ref, k_ref, v_ref):
    xn = _rms(x_ref[...], anw_ref[...]).astype(jnp.bfloat16)
    cc = jnp.concatenate([c_ref[...]] * (_DIM // 128), axis=1)
    d1 = jnp.concatenate([d1_ref[...]] * (_DIM // 128), axis=1)
    d2 = jnp.concatenate([d2_ref[...]] * (_DIM // 128), axis=1)
    q = _dot_t(xn, wq_ref[...])
    k = _dot_t(xn, wk_ref[...])
    q_ref[...] = _rope(q, cc, d1, d2).astype(jnp.bfloat16)
    k_ref[...] = _rope(k, cc, d1, d2).astype(jnp.bfloat16)
    v_ref[...] = _dot_t(xn, wv_ref[...]).astype(jnp.bfloat16)


def _qkv_call(x2, c_tab, d1_tab, d2_tab, anw, wqb, wkb, wvb):
    n = _S // _SBLK
    blk_s = pl.BlockSpec((_SBLK, _DIM), lambda i: (i, 0))
    blk_t = pl.BlockSpec((_SBLK, 128), lambda i: (i, 0))
    w_full = pl.BlockSpec((_DIM, _DIM), lambda i: (0, 0))
    return pl.pallas_call(
        _qkv_body,
        grid=(n,),
        in_specs=[blk_s, blk_t, blk_t, blk_t,
                  pl.BlockSpec((1, _DIM), lambda i: (0, 0)),
                  w_full, w_full, w_full],
        out_specs=[blk_s, blk_s, blk_s],
        out_shape=[jax.ShapeDtypeStruct((_S, _DIM), jnp.bfloat16)] * 3,
    )(x2, c_tab, d1_tab, d2_tab, anw, wqb, wkb, wvb)


def _att_body(q_ref, k_ref, v_ref, o_ref):
    ones = jnp.ones((_S, 1), jnp.bfloat16)
    outs = []
    for p in range(2):
        q = q_ref[:, p * _HD:(p + 1) * _HD]
        k = k_ref[:, p * _HD:(p + 1) * _HD]
        v = v_ref[:, p * _HD:(p + 1) * _HD]
        s = _dot_t(q, k) * (1.0 / math.sqrt(_HD))
        eb = jnp.exp(s).astype(jnp.bfloat16)
        vv = jnp.concatenate([v, ones], axis=1)
        acc = jnp.dot(eb, vv, preferred_element_type=jnp.float32)
        outs.append(acc[:, :_HD] * (1.0 / acc[:, _HD:_HD + 1]))
    o_ref[...] = jnp.concatenate(outs, axis=1).astype(jnp.bfloat16)


def _att_call(q, k, v):
    grid = (_H // 2, _S // _ABLK)
    qo_spec = pl.BlockSpec((_ABLK, 2 * _HD), lambda h, i: (i, h))
    kv_spec = pl.BlockSpec((_S, 2 * _HD), lambda h, i: (0, h))
    return pl.pallas_call(
        _att_body,
        grid=grid,
        in_specs=[qo_spec, kv_spec, kv_spec],
        out_specs=qo_spec,
        out_shape=jax.ShapeDtypeStruct((_S, _DIM), jnp.bfloat16),
    )(q, k, v)


def _post_body(x_ref, o_ref, wo_ref, fnw_ref, gw_ref, h_ref, xf_ref, gs_ref):
    h = x_ref[...] + _dot_t(o_ref[...], wo_ref[...])
    xf = _rms(h, fnw_ref[...])
    h_ref[...] = h
    xf_ref[...] = xf
    gs_ref[...] = _dot_t(xf, gw_ref[...])


def _post_call(x2, o, wob, fnw, gate_w):
    n = _S // _SBLK
    blk_s = pl.BlockSpec((_SBLK, _DIM), lambda i: (i, 0))
    w_full = pl.BlockSpec((_DIM, _DIM), lambda i: (0, 0))
    return pl.pallas_call(
        _post_body,
        grid=(n,),
        in_specs=[blk_s, blk_s, w_full,
                  pl.BlockSpec((1, _DIM), lambda i: (0, 0)),
                  pl.BlockSpec((_E, _DIM), lambda i: (0, 0))],
        out_specs=[blk_s, blk_s, pl.BlockSpec((_SBLK, _E), lambda i: (i, 0))],
        out_shape=[jax.ShapeDtypeStruct((_S, _DIM), jnp.float32),
                   jax.ShapeDtypeStruct((_S, _DIM), jnp.float32),
                   jax.ShapeDtypeStruct((_S, _E), jnp.float32)],
    )(x2, o, wob, fnw, gate_w)


def _shared_body(xf_ref, sw1_ref, sw3_ref, sw2_ref, sh_ref):
    xfb = xf_ref[...].astype(jnp.bfloat16)
    g1 = _dot_t(xfb, sw1_ref[...])
    g3 = _dot_t(xfb, sw3_ref[...])
    gg = (g1 * jax.nn.sigmoid(g1) * g3).astype(jnp.bfloat16)
    sh_ref[...] = _dot_t(gg, sw2_ref[...])


def _shared_call(xf, sw1b, sw3b, sw2b):
    n = _S // _SBLK
    blk_s = pl.BlockSpec((_SBLK, _DIM), lambda i: (i, 0))
    w_full = pl.BlockSpec((_DIM, _DIM), lambda i: (0, 0))
    return pl.pallas_call(
        _shared_body,
        grid=(n,),
        in_specs=[blk_s, w_full, w_full, w_full],
        out_specs=blk_s,
        out_shape=jax.ShapeDtypeStruct((_S, _DIM), jnp.float32),
    )(xf, sw1b, sw3b, sw2b)


def _route_body(gs_ref, posi_ref, ew_ref, cnt_ref, st_ref):
    rows = _NA // 8
    gs = gs_ref[...]
    m = jnp.max(gs, axis=-1, keepdims=True)
    eg = jnp.exp(gs - m)
    probs = eg / jnp.sum(eg, axis=-1, keepdims=True)
    lane = lax.broadcasted_iota(jnp.int32, (_S, _E), 1)
    m1 = jnp.max(probs, axis=-1, keepdims=True)
    i1 = jnp.min(jnp.where(probs == m1, lane, _E), axis=-1, keepdims=True)
    oh1 = lane == i1
    probs2 = jnp.where(oh1, -jnp.inf, probs)
    m2 = jnp.max(probs2, axis=-1, keepdims=True)
    i2 = jnp.min(jnp.where(probs2 == m2, lane, _E), axis=-1, keepdims=True)
    oh2 = lane == i2

    wsum = m1 + m2 + 1e-9
    ew_ref[...] = jnp.concatenate([m1 / wsum, m2 / wsum], axis=1)

    oh = jnp.concatenate([oh1, oh2], axis=0).astype(jnp.float32)
    counts = jnp.sum(oh, axis=0, keepdims=True)
    cnt_ref[...] = counts.astype(jnp.int32)

    pc = jnp.ceil(counts / _FBLK) * _FBLK
    tri = (lax.broadcasted_iota(jnp.int32, (_E, _E), 0)
           < lax.broadcasted_iota(jnp.int32, (_E, _E), 1)).astype(jnp.float32)
    po = jnp.dot(pc, tri, preferred_element_type=jnp.float32)

    rank = jnp.concatenate([jnp.zeros((1, _E), jnp.float32), oh[:-1]], axis=0)
    d = 1
    while d < _NA:
        rank = rank + jnp.concatenate(
            [jnp.zeros((d, _E), jnp.float32), rank[:-d]], axis=0)
        d *= 2
    posb = jnp.sum(oh * (po + rank), axis=1, keepdims=True)
    posi_ref[...] = posb.astype(jnp.int32)

    cols = 512
    tok = lax.broadcasted_iota(jnp.int32, (3, _NA), 1) % _S
    lrow = lax.broadcasted_iota(jnp.int32, (3, _NA), 0)
    lhs = jnp.where(lrow == 0, tok // 128,
                    jnp.where(lrow == 1, tok % 128, 1)
                    ).astype(jnp.bfloat16)
    for cb in range(_NPAD // cols):
        sg = (lax.broadcasted_iota(jnp.int32, (_NA, cols), 1)
              + cb * cols).astype(jnp.float32)
        eqb = (posb == sg).astype(jnp.bfloat16)
        red = jnp.dot(lhs, eqb, preferred_element_type=jnp.float32)
        st = red[0:1] * 128.0 + red[1:2]
        hit = red[2:3]
        fb = (lax.broadcasted_iota(jnp.int32, (1, cols), 1) + cb * cols) % _S
        st_ref[:, cb * cols:(cb + 1) * cols] = jnp.where(
            hit > 0.0, st, fb.astype(jnp.float32)).astype(jnp.int32)


def _route_call(gs):
    return pl.pallas_call(
        _route_body,
        grid=(1,),
        in_specs=[pl.BlockSpec((_S, _E), lambda b: (0, 0))],
        out_specs=[pl.BlockSpec((_NA, 1), lambda b: (0, 0)),
                   pl.BlockSpec((_S, 2), lambda b: (0, 0)),
                   pl.BlockSpec((1, _E), lambda b: (0, 0)),
                   pl.BlockSpec((1, _NPAD), lambda b: (0, 0))],
        out_shape=[jax.ShapeDtypeStruct((_NA, 1), jnp.int32),
                   jax.ShapeDtypeStruct((_S, 2), jnp.float32),
                   jax.ShapeDtypeStruct((1, _E), jnp.int32),
                   jax.ShapeDtypeStruct((1, _NPAD), jnp.int32)],
    )(gs)


def _sc_gather(table, idx, nrows_out):
    nw = 32
    per_w = nrows_out // nw
    ch = 32
    nch = per_w // ch
    mesh = plsc.VectorSubcoreMesh(core_axis_name="c", subcore_axis_name="s")

    @functools.partial(
        pl.kernel,
        out_type=jax.ShapeDtypeStruct((nrows_out, _DIM), jnp.float32),
        mesh=mesh,
        scratch_types=[pltpu.VMEM((per_w,), jnp.int32),
                       pltpu.VMEM((ch, _DIM), jnp.float32),
                       pltpu.VMEM((ch, _DIM), jnp.float32),
                       pltpu.SemaphoreType.DMA,
                       pltpu.SemaphoreType.DMA],
    )
    def gk(table_hbm, idx_hbm, out_hbm, idx_v, rows0, rows1, sem0, sem1):
        wid = lax.axis_index("s") * 2 + lax.axis_index("c")
        base = wid * per_w
        pltpu.sync_copy(idx_hbm.at[pl.ds(base, per_w)], idx_v)
        bufs = (rows0, rows1)
        sems = (sem0, sem1)
        pend = pltpu.async_copy(table_hbm.at[idx_v.at[pl.ds(0, ch)]],
                                bufs[0], sems[0])
        for ci in range(1, nch):
            nxt = pltpu.async_copy(
                table_hbm.at[idx_v.at[pl.ds(ci * ch, ch)]],
                bufs[ci % 2], sems[ci % 2])
            pend.wait()
            pltpu.sync_copy(bufs[(ci - 1) % 2],
                            out_hbm.at[pl.ds(base + (ci - 1) * ch, ch)])
            pend = nxt
        pend.wait()
        pltpu.sync_copy(bufs[(nch - 1) % 2],
                        out_hbm.at[pl.ds(base + (nch - 1) * ch, ch)])

    return gk(table, idx)


def _ffn_body(sp_ref, xs_ref, w1_ref, w2_ref, w3_ref, eo_ref):
    b = pl.program_id(0)

    @pl.when(b < sp_ref[_NFB])
    def _():
        xsb = xs_ref[...].astype(jnp.bfloat16)
        h1 = _dot_t(xsb, w1_ref[0])
        h3 = _dot_t(xsb, w3_ref[0])
        gg = (h1 * jax.nn.sigmoid(h1) * h3).astype(jnp.bfloat16)
        eo_ref[...] = _dot_t(gg, w2_ref[0])


def _ffn_call(sp, xs, w1b, w2b, w3b):
    def _row_idx(b, sp_ref):
        return (jnp.minimum(b, sp_ref[_NFB] - 1), 0)

    grid_spec = pltpu.PrefetchScalarGridSpec(
        num_scalar_prefetch=1,
        grid=(_NFB,),
        in_specs=[
            pl.BlockSpec((_FBLK, _DIM), _row_idx),
            pl.BlockSpec((1, _HID, _DIM), lambda b, sp: (sp[b], 0, 0)),
            pl.BlockSpec((1, _DIM, _HID), lambda b, sp: (sp[b], 0, 0)),
            pl.BlockSpec((1, _HID, _DIM), lambda b, sp: (sp[b], 0, 0)),
        ],
        out_specs=pl.BlockSpec((_FBLK, _DIM), _row_idx),
    )
    return pl.pallas_call(
        _ffn_body,
        grid_spec=grid_spec,
        out_shape=jax.ShapeDtypeStruct((_NPAD, _DIM), jnp.float32),
    )(sp, xs, w1b, w2b, w3b)


def _comb_body(h_ref, sh_ref, ew_ref, y0_ref, y1_ref, out_ref):
    w = ew_ref[...]
    out_ref[...] = (h_ref[...] + sh_ref[...] + w[:, 0:1] * y0_ref[...]
                    + w[:, 1:2] * y1_ref[...])


def _comb_call(h, sh, ew, yg):
    n = _S // _SBLK
    blk_s = pl.BlockSpec((_SBLK, _DIM), lambda i: (i, 0))
    return pl.pallas_call(
        _comb_body,
        grid=(n,),
        in_specs=[blk_s, blk_s,
                  pl.BlockSpec((_SBLK, 2), lambda i: (i, 0)),
                  pl.BlockSpec((_SBLK, _DIM), lambda i: (i, 0)),
                  pl.BlockSpec((_SBLK, _DIM), lambda i: (n + i, 0))],
        out_specs=blk_s,
        out_shape=jax.ShapeDtypeStruct((_S, _DIM), jnp.float32),
    )(h, sh, ew, yg, yg)


def kernel(x, freqs_cis, att_norm_w, wq, wk, wv, wo, ffn_norm_w, gate_w,
           w1, w2, w3, sw1, sw2, sw3):
    x2 = x.reshape(_S, _DIM)
    fr, fi = freqs_cis[..., 0], freqs_cis[..., 1]
    frw = jnp.tile(jnp.repeat(fr, 2, axis=1), (1, 128 // _HD))
    fiw = jnp.tile(jnp.repeat(fi, 2, axis=1), (1, 128 // _HD))
    even = (jnp.arange(128) % 2 == 0)[None, :]
    c_tab = frw
    d1_tab = jnp.where(even, -fiw, 0.0)
    d2_tab = jnp.where(even, 0.0, fiw)

    bf = jnp.bfloat16
    q, k, v = _qkv_call(x2, c_tab, d1_tab, d2_tab, att_norm_w.reshape(1, _DIM),
                        wq.astype(bf), wk.astype(bf), wv.astype(bf))
    o = _att_call(q, k, v)
    h, xf, gs = _post_call(x2, o, wo.astype(bf),
                           ffn_norm_w.reshape(1, _DIM), gate_w)
    posi, ew, cnt, st = _route_call(gs)
    st_i = st.reshape(_NPAD)

    counts = cnt.reshape(_E)
    pc = ((counts + _FBLK - 1) // _FBLK) * _FBLK
    po = jnp.concatenate([jnp.zeros((1,), jnp.int32), jnp.cumsum(pc)[:-1]])
    ends = po + pc
    nblk = jnp.sum(pc) // _FBLK
    bidx = jnp.arange(_NFB, dtype=jnp.int32)
    beid_raw = jnp.minimum(
        jnp.sum((bidx[:, None] * _FBLK >= ends[None, :]).astype(jnp.int32),
                axis=1), _E - 1)
    fill = beid_raw[nblk - 1]
    beid = jnp.where(bidx < nblk, beid_raw, fill)
    sp = jnp.concatenate([beid, nblk[None]]).astype(jnp.int32)

    xs = _sc_gather(xf, st_i, _NPAD)
    sh = _shared_call(xf, sw1.astype(bf), sw3.astype(bf), sw2.astype(bf))
    eo = _ffn_call(sp, xs, w1.astype(bf), w2.astype(bf), w3.astype(bf))
    yg = _sc_gather(eo, posi.reshape(_NA), _NA)
    out = _comb_call(h, sh, ew, yg)
    return out.reshape(_B, _S, _DIM)


# speedup vs baseline: 1.1302x; 1.0038x over previous
import functools
import math

import jax
import jax.numpy as jnp
from jax import lax
from jax.experimental import pallas as pl
from jax.experimental.pallas import tpu as pltpu
from jax.experimental.pallas import tpu_sc as plsc

_B, _S, _DIM, _H, _HD = 1, 2048, 1024, 16, 64
_E, _TOPK, _HID, _SHID = 8, 2, 1024, 1024
_EPS = 1e-6

_SBLK = 512
_ABLK = 1024
_FBLK = 128
_NA = _TOPK * _S
_NPAD = _NA + _E * _FBLK
_NFB = _NPAD // _FBLK


def _rms(x, w):
    return x * lax.rsqrt(jnp.mean(x * x, axis=-1, keepdims=True) + _EPS) * w


def _dot_t(a, b):
    return lax.dot_general(a, b, (((1,), (1,)), ((), ())),
                           preferred_element_type=jnp.float32)


def _rope(y, cc, d1, d2):
    left = jnp.concatenate([y[:, 1:], y[:, :1]], axis=1)
    right = jnp.concatenate([y[:, -1:], y[:, :-1]], axis=1)
    return y * cc + left * d1 + right * d2


def _qkv_body(x_ref, c_ref, d1_ref, d2_ref, anw_ref, wq_ref, wk_ref, wv_ref,
              q_ref, k_ref, v_ref):
    xn = _rms(x_ref[...], anw_ref[...]).astype(jnp.bfloat16)
    cc = jnp.concatenate([c_ref[...]] * (_DIM // 128), axis=1)
    d1 = jnp.concatenate([d1_ref[...]] * (_DIM // 128), axis=1)
    d2 = jnp.concatenate([d2_ref[...]] * (_DIM // 128), axis=1)
    q = _dot_t(xn, wq_ref[...])
    k = _dot_t(xn, wk_ref[...])
    q_ref[...] = _rope(q, cc, d1, d2).astype(jnp.bfloat16)
    k_ref[...] = _rope(k, cc, d1, d2).astype(jnp.bfloat16)
    v_ref[...] = _dot_t(xn, wv_ref[...]).astype(jnp.bfloat16)


def _qkv_call(x2, c_tab, d1_tab, d2_tab, anw, wqb, wkb, wvb):
    n = _S // _SBLK
    blk_s = pl.BlockSpec((_SBLK, _DIM), lambda i: (i, 0))
    blk_t = pl.BlockSpec((_SBLK, 128), lambda i: (i, 0))
    w_full = pl.BlockSpec((_DIM, _DIM), lambda i: (0, 0))
    return pl.pallas_call(
        _qkv_body,
        grid=(n,),
        in_specs=[blk_s, blk_t, blk_t, blk_t,
                  pl.BlockSpec((1, _DIM), lambda i: (0, 0)),
                  w_full, w_full, w_full],
        out_specs=[blk_s, blk_s, blk_s],
        out_shape=[jax.ShapeDtypeStruct((_S, _DIM), jnp.bfloat16)] * 3,
    )(x2, c_tab, d1_tab, d2_tab, anw, wqb, wkb, wvb)


def _att_body(q_ref, k_ref, v_ref, o_ref):
    ones = jnp.ones((_S, 1), jnp.bfloat16)
    outs = []
    for p in range(2):
        q = q_ref[:, p * _HD:(p + 1) * _HD]
        k = k_ref[:, p * _HD:(p + 1) * _HD]
        v = v_ref[:, p * _HD:(p + 1) * _HD]
        s = _dot_t(q, k) * (1.0 / math.sqrt(_HD))
        eb = jnp.exp(s).astype(jnp.bfloat16)
        vv = jnp.concatenate([v, ones], axis=1)
        acc = jnp.dot(eb, vv, preferred_element_type=jnp.float32)
        outs.append(acc[:, :_HD] * (1.0 / acc[:, _HD:_HD + 1]))
    o_ref[...] = jnp.concatenate(outs, axis=1).astype(jnp.bfloat16)


def _att_call(q, k, v):
    grid = (_H // 2, _S // _ABLK)
    qo_spec = pl.BlockSpec((_ABLK, 2 * _HD), lambda h, i: (i, h))
    kv_spec = pl.BlockSpec((_S, 2 * _HD), lambda h, i: (0, h))
    return pl.pallas_call(
        _att_body,
        grid=grid,
        in_specs=[qo_spec, kv_spec, kv_spec],
        out_specs=qo_spec,
        out_shape=jax.ShapeDtypeStruct((_S, _DIM), jnp.bfloat16),
    )(q, k, v)


def _post_body(x_ref, o_ref, wo_ref, fnw_ref, gw_ref, sw1_ref, sw3_ref,
               sw2_ref, hs_ref, xf_ref, gs_ref):
    h = x_ref[...] + _dot_t(o_ref[...], wo_ref[...])
    xf = _rms(h, fnw_ref[...])
    xf_ref[...] = xf
    gs_ref[...] = _dot_t(xf, gw_ref[...])
    xfb = xf.astype(jnp.bfloat16)
    g1 = _dot_t(xfb, sw1_ref[...])
    g3 = _dot_t(xfb, sw3_ref[...])
    gg = (g1 * jax.nn.sigmoid(g1) * g3).astype(jnp.bfloat16)
    hs_ref[...] = h + _dot_t(gg, sw2_ref[...])


def _post_call(x2, o, wob, fnw, gate_w, sw1b, sw3b, sw2b):
    n = _S // _SBLK
    blk_s = pl.BlockSpec((_SBLK, _DIM), lambda i: (i, 0))
    w_full = pl.BlockSpec((_DIM, _DIM), lambda i: (0, 0))
    return pl.pallas_call(
        _post_body,
        grid=(n,),
        in_specs=[blk_s, blk_s, w_full,
                  pl.BlockSpec((1, _DIM), lambda i: (0, 0)),
                  pl.BlockSpec((_E, _DIM), lambda i: (0, 0)),
                  w_full, w_full, w_full],
        out_specs=[blk_s, blk_s, pl.BlockSpec((_SBLK, _E), lambda i: (i, 0))],
        out_shape=[jax.ShapeDtypeStruct((_S, _DIM), jnp.float32),
                   jax.ShapeDtypeStruct((_S, _DIM), jnp.float32),
                   jax.ShapeDtypeStruct((_S, _E), jnp.float32)],
    )(x2, o, wob, fnw, gate_w, sw1b, sw3b, sw2b)


def _route_body(gs_ref, posi_ref, ew_ref, cnt_ref, st_ref):
    rows = _NA // 8
    gs = gs_ref[...]
    m = jnp.max(gs, axis=-1, keepdims=True)
    eg = jnp.exp(gs - m)
    probs = eg / jnp.sum(eg, axis=-1, keepdims=True)
    lane = lax.broadcasted_iota(jnp.int32, (_S, _E), 1)
    m1 = jnp.max(probs, axis=-1, keepdims=True)
    i1 = jnp.min(jnp.where(probs == m1, lane, _E), axis=-1, keepdims=True)
    oh1 = lane == i1
    probs2 = jnp.where(oh1, -jnp.inf, probs)
    m2 = jnp.max(probs2, axis=-1, keepdims=True)
    i2 = jnp.min(jnp.where(probs2 == m2, lane, _E), axis=-1, keepdims=True)
    oh2 = lane == i2

    wsum = m1 + m2 + 1e-9
    ew_ref[...] = jnp.concatenate([m1 / wsum, m2 / wsum], axis=1)

    oh = jnp.concatenate([oh1, oh2], axis=0).astype(jnp.float32)
    counts = jnp.sum(oh, axis=0, keepdims=True)
    cnt_ref[...] = counts.astype(jnp.int32)

    pc = jnp.ceil(counts / _FBLK) * _FBLK
    tri = (lax.broadcasted_iota(jnp.int32, (_E, _E), 0)
           < lax.broadcasted_iota(jnp.int32, (_E, _E), 1)).astype(jnp.float32)
    po = jnp.dot(pc, tri, preferred_element_type=jnp.float32)

    rank = jnp.concatenate([jnp.zeros((1, _E), jnp.float32), oh[:-1]], axis=0)
    d = 1
    while d < _NA:
        rank = rank + jnp.concatenate(
            [jnp.zeros((d, _E), jnp.float32), rank[:-d]], axis=0)
        d *= 2
    posb = jnp.sum(oh * (po + rank), axis=1, keepdims=True)
    posi_ref[...] = posb.astype(jnp.int32)

    cols = 512
    tok = lax.broadcasted_iota(jnp.int32, (3, _NA), 1) % _S
    lrow = lax.broadcasted_iota(jnp.int32, (3, _NA), 0)
    lhs = jnp.where(lrow == 0, tok // 128,
                    jnp.where(lrow == 1, tok % 128, 1)
                    ).astype(jnp.bfloat16)
    for cb in range(_NPAD // cols):
        sg = (lax.broadcasted_iota(jnp.int32, (_NA, cols), 1)
              + cb * cols).astype(jnp.float32)
        eqb = (posb == sg).astype(jnp.bfloat16)
        red = jnp.dot(lhs, eqb, preferred_element_type=jnp.float32)
        st = red[0:1] * 128.0 + red[1:2]
        hit = red[2:3]
        fb = (lax.broadcasted_iota(jnp.int32, (1, cols), 1) + cb * cols) % _S
        st_ref[:, cb * cols:(cb + 1) * cols] = jnp.where(
            hit > 0.0, st, fb.astype(jnp.float32)).astype(jnp.int32)


def _route_call(gs):
    return pl.pallas_call(
        _route_body,
        grid=(1,),
        in_specs=[pl.BlockSpec((_S, _E), lambda b: (0, 0))],
        out_specs=[pl.BlockSpec((_NA, 1), lambda b: (0, 0)),
                   pl.BlockSpec((_S, 2), lambda b: (0, 0)),
                   pl.BlockSpec((1, _E), lambda b: (0, 0)),
                   pl.BlockSpec((1, _NPAD), lambda b: (0, 0))],
        out_shape=[jax.ShapeDtypeStruct((_NA, 1), jnp.int32),
                   jax.ShapeDtypeStruct((_S, 2), jnp.float32),
                   jax.ShapeDtypeStruct((1, _E), jnp.int32),
                   jax.ShapeDtypeStruct((1, _NPAD), jnp.int32)],
    )(gs)


def _sc_gather(table, idx, nrows_out):
    nw = 32
    per_w = nrows_out // nw
    ch = 32
    nch = per_w // ch
    mesh = plsc.VectorSubcoreMesh(core_axis_name="c", subcore_axis_name="s")

    @functools.partial(
        pl.kernel,
        out_type=jax.ShapeDtypeStruct((nrows_out, _DIM), jnp.float32),
        mesh=mesh,
        scratch_types=[pltpu.VMEM((per_w,), jnp.int32),
                       pltpu.VMEM((ch, _DIM), jnp.float32),
                       pltpu.VMEM((ch, _DIM), jnp.float32),
                       pltpu.SemaphoreType.DMA,
                       pltpu.SemaphoreType.DMA],
    )
    def gk(table_hbm, idx_hbm, out_hbm, idx_v, rows0, rows1, sem0, sem1):
        wid = lax.axis_index("s") * 2 + lax.axis_index("c")
        base = wid * per_w
        pltpu.sync_copy(idx_hbm.at[pl.ds(base, per_w)], idx_v)
        bufs = (rows0, rows1)
        sems = (sem0, sem1)
        pend = pltpu.async_copy(table_hbm.at[idx_v.at[pl.ds(0, ch)]],
                                bufs[0], sems[0])
        for ci in range(1, nch):
            nxt = pltpu.async_copy(
                table_hbm.at[idx_v.at[pl.ds(ci * ch, ch)]],
                bufs[ci % 2], sems[ci % 2])
            pend.wait()
            pltpu.sync_copy(bufs[(ci - 1) % 2],
                            out_hbm.at[pl.ds(base + (ci - 1) * ch, ch)])
            pend = nxt
        pend.wait()
        pltpu.sync_copy(bufs[(nch - 1) % 2],
                        out_hbm.at[pl.ds(base + (nch - 1) * ch, ch)])

    return gk(table, idx)


def _ffn_body(sp_ref, xs_ref, w1_ref, w2_ref, w3_ref, eo_ref):
    b = pl.program_id(0)

    @pl.when(b < sp_ref[_NFB])
    def _():
        xsb = xs_ref[...].astype(jnp.bfloat16)
        h1 = _dot_t(xsb, w1_ref[0])
        h3 = _dot_t(xsb, w3_ref[0])
        gg = (h1 * jax.nn.sigmoid(h1) * h3).astype(jnp.bfloat16)
        eo_ref[...] = _dot_t(gg, w2_ref[0])


def _ffn_call(sp, xs, w1b, w2b, w3b):
    def _row_idx(b, sp_ref):
        return (jnp.minimum(b, sp_ref[_NFB] - 1), 0)

    grid_spec = pltpu.PrefetchScalarGridSpec(
        num_scalar_prefetch=1,
        grid=(_NFB,),
        in_specs=[
            pl.BlockSpec((_FBLK, _DIM), _row_idx),
            pl.BlockSpec((1, _HID, _DIM), lambda b, sp: (sp[b], 0, 0)),
            pl.BlockSpec((1, _DIM, _HID), lambda b, sp: (sp[b], 0, 0)),
            pl.BlockSpec((1, _HID, _DIM), lambda b, sp: (sp[b], 0, 0)),
        ],
        out_specs=pl.BlockSpec((_FBLK, _DIM), _row_idx),
    )
    return pl.pallas_call(
        _ffn_body,
        grid_spec=grid_spec,
        out_shape=jax.ShapeDtypeStruct((_NPAD, _DIM), jnp.float32),
    )(sp, xs, w1b, w2b, w3b)


def _comb_body(hs_ref, ew_ref, y0_ref, y1_ref, out_ref):
    w = ew_ref[...]
    out_ref[...] = (hs_ref[...] + w[:, 0:1] * y0_ref[...]
                    + w[:, 1:2] * y1_ref[...])


def _comb_call(hs, ew, yg):
    n = _S // _SBLK
    blk_s = pl.BlockSpec((_SBLK, _DIM), lambda i: (i, 0))
    return pl.pallas_call(
        _comb_body,
        grid=(n,),
        in_specs=[blk_s,
                  pl.BlockSpec((_SBLK, 2), lambda i: (i, 0)),
                  pl.BlockSpec((_SBLK, _DIM), lambda i: (i, 0)),
                  pl.BlockSpec((_SBLK, _DIM), lambda i: (n + i, 0))],
        out_specs=blk_s,
        out_shape=jax.ShapeDtypeStruct((_S, _DIM), jnp.float32),
    )(hs, ew, yg, yg)


def kernel(x, freqs_cis, att_norm_w, wq, wk, wv, wo, ffn_norm_w, gate_w,
           w1, w2, w3, sw1, sw2, sw3):
    x2 = x.reshape(_S, _DIM)
    fr, fi = freqs_cis[..., 0], freqs_cis[..., 1]
    frw = jnp.tile(jnp.repeat(fr, 2, axis=1), (1, 128 // _HD))
    fiw = jnp.tile(jnp.repeat(fi, 2, axis=1), (1, 128 // _HD))
    even = (jnp.arange(128) % 2 == 0)[None, :]
    c_tab = frw
    d1_tab = jnp.where(even, -fiw, 0.0)
    d2_tab = jnp.where(even, 0.0, fiw)

    bf = jnp.bfloat16
    q, k, v = _qkv_call(x2, c_tab, d1_tab, d2_tab, att_norm_w.reshape(1, _DIM),
                        wq.astype(bf), wk.astype(bf), wv.astype(bf))
    o = _att_call(q, k, v)
    hs, xf, gs = _post_call(x2, o, wo.astype(bf),
                            ffn_norm_w.reshape(1, _DIM), gate_w,
                            sw1.astype(bf), sw3.astype(bf), sw2.astype(bf))
    posi, ew, cnt, st = _route_call(gs)
    st_i = st.reshape(_NPAD)

    counts = cnt.reshape(_E)
    pc = ((counts + _FBLK - 1) // _FBLK) * _FBLK
    po = jnp.concatenate([jnp.zeros((1,), jnp.int32), jnp.cumsum(pc)[:-1]])
    ends = po + pc
    nblk = jnp.sum(pc) // _FBLK
    bidx = jnp.arange(_NFB, dtype=jnp.int32)
    beid_raw = jnp.minimum(
        jnp.sum((bidx[:, None] * _FBLK >= ends[None, :]).astype(jnp.int32),
                axis=1), _E - 1)
    fill = beid_raw[nblk - 1]
    beid = jnp.where(bidx < nblk, beid_raw, fill)
    sp = jnp.concatenate([beid, nblk[None]]).astype(jnp.int32)

    xs = _sc_gather(xf, st_i, _NPAD)
    eo = _ffn_call(sp, xs, w1.astype(bf), w2.astype(bf), w3.astype(bf))
    yg = _sc_gather(eo, posi.reshape(_NA), _NA)
    out = _comb_call(hs, ew, yg)
    return out.reshape(_B, _S, _DIM)
